# Initial kernel scaffold; baseline (speedup 1.0000x reference)
import jax
import jax.numpy as jnp
from jax.experimental import pallas as pl


def kernel(x, codebook):
    raise NotImplementedError("write your pallas kernel here")


# speedup vs baseline: 1.7302x; 1.7302x over previous
import functools

import jax
import jax.numpy as jnp
from jax.experimental import pallas as pl
from jax.experimental.pallas import tpu as pltpu

NUM_EMB = 1024
EMB_DIM = 64


def _vq_kernel(x_ref, cb_ref, s2_ref, out_ref, codes_ref, loss_ref):
    b = pl.program_id(0)

    x_blk = x_ref[0]
    cb = cb_ref[...]
    flat = x_blk.T

    m = jax.lax.dot_general(
        flat, cb,
        dimension_numbers=(((1,), (1,)), ((), ())),
        preferred_element_type=jnp.float32,
    )
    s1 = jnp.sum(flat * flat, axis=1, keepdims=True)
    s2 = s2_ref[...]
    d2 = (s1 - 2.0 * m) + s2

    minv = jnp.min(d2, axis=1, keepdims=True)
    iota_l = jax.lax.broadcasted_iota(jnp.int32, (NUM_EMB, NUM_EMB), 1)
    codes = jnp.min(jnp.where(d2 == minv, iota_l, NUM_EMB), axis=1).astype(jnp.int32)
    codes_ref[0, 0, :] = codes

    iota_k = jax.lax.broadcasted_iota(jnp.int32, (NUM_EMB, NUM_EMB), 0)
    onehot = (iota_k == codes[None, :]).astype(jnp.float32)
    q_t = jax.lax.dot_general(
        cb, onehot,
        dimension_numbers=(((0,), (0,)), ((), ())),
        preferred_element_type=jnp.float32,
    )
    out_ref[0] = q_t

    diff = q_t - x_blk
    blk_loss = jnp.sum(diff * diff)

    @pl.when(b == 0)
    def _init():
        loss_ref[0, 0] = 0.0

    loss_ref[0, 0] += blk_loss


@functools.partial(jax.jit, static_argnames=())
def kernel(x, codebook):
    B, D, H, W = x.shape
    HW = H * W
    x3 = x.reshape(B, D, HW)
    s2 = jnp.sum(codebook ** 2, axis=1)[None, :]

    out, codes3, loss_sum = pl.pallas_call(
        _vq_kernel,
        grid=(B,),
        in_specs=[
            pl.BlockSpec((1, D, HW), lambda b: (b, 0, 0)),
            pl.BlockSpec((NUM_EMB, EMB_DIM), lambda b: (0, 0)),
            pl.BlockSpec((1, NUM_EMB), lambda b: (0, 0)),
        ],
        out_specs=[
            pl.BlockSpec((1, D, HW), lambda b: (b, 0, 0)),
            pl.BlockSpec((1, 1, HW), lambda b: (b, 0, 0)),
            pl.BlockSpec(memory_space=pltpu.SMEM),
        ],
        out_shape=[
            jax.ShapeDtypeStruct((B, D, HW), jnp.float32),
            jax.ShapeDtypeStruct((B, 1, HW), jnp.int32),
            jax.ShapeDtypeStruct((1, 1), jnp.float32),
        ],
    )(x3, codebook, s2)

    quantized_x = out.reshape(B, D, H, W)
    codes = codes3.reshape(B, HW)
    latent_loss = 2.0 * loss_sum[0, 0] / (B * HW * D)
    return quantized_x, codes, latent_loss
